# baseline (speedup 1.0000x reference)
import functools

import jax
import jax.numpy as jnp
from jax import lax
from jax.experimental import pallas as pl
from jax.experimental.pallas import tpu as pltpu
from jax.experimental.pallas import tpu_sc as plsc

_NUM_BANDS = 64
_EMBED_DIM = 128
_B = 4096
_NC = 2
_NS = 16
_NW = _NC * _NS
_BPW = _B // _NW
_REP = 2
_NFULL = _BPW // _REP
_TAIL = _BPW % _REP

_mesh = plsc.VectorSubcoreMesh(core_axis_name="c", subcore_axis_name="s")


@functools.partial(
    pl.kernel,
    mesh=_mesh,
    out_type=jax.ShapeDtypeStruct((_B, _NUM_BANDS, _EMBED_DIM), jnp.float32),
    scratch_types=[
        pltpu.VMEM((_REP, _NUM_BANDS, _EMBED_DIM), jnp.float32),
        pltpu.SemaphoreType.DMA,
        pltpu.SemaphoreType.DMA,
    ],
)
def _broadcast_sc(table_hbm, out_hbm, buf, sem, stage_sem):
    wid = lax.axis_index("s") * _NC + lax.axis_index("c")
    base = wid * _BPW
    stage = [pltpu.async_copy(table_hbm, buf.at[r], stage_sem) for r in range(_REP)]
    for c in stage:
        c.wait()
    copies = []
    for i in range(_NFULL):
        copies.append(
            pltpu.async_copy(buf, out_hbm.at[pl.ds(base + i * _REP, _REP)], sem)
        )
    if _TAIL:
        copies.append(
            pltpu.async_copy(
                buf.at[pl.ds(0, _TAIL)],
                out_hbm.at[pl.ds(base + _NFULL * _REP, _TAIL)],
                sem,
            )
        )
    for c in copies:
        c.wait()


def kernel(embedding_weight, batch_size):
    del batch_size
    return _broadcast_sc(embedding_weight)


# speedup vs baseline: 1.0127x; 1.0127x over previous
import functools

import jax
import jax.numpy as jnp
from jax import lax
from jax.experimental import pallas as pl
from jax.experimental.pallas import tpu as pltpu
from jax.experimental.pallas import tpu_sc as plsc

_NUM_BANDS = 64
_EMBED_DIM = 128
_B = 4096
_NC = 2
_NS = 16
_NW = _NC * _NS
_BPW = _B // _NW
_REP = 1
_NFULL = _BPW // _REP
_TAIL = _BPW % _REP

_mesh = plsc.VectorSubcoreMesh(core_axis_name="c", subcore_axis_name="s")


@functools.partial(
    pl.kernel,
    mesh=_mesh,
    out_type=jax.ShapeDtypeStruct((_B, _NUM_BANDS, _EMBED_DIM), jnp.float32),
    scratch_types=[
        pltpu.VMEM((_REP, _NUM_BANDS, _EMBED_DIM), jnp.float32),
        pltpu.SemaphoreType.DMA,
        pltpu.SemaphoreType.DMA,
    ],
)
def _broadcast_sc(table_hbm, out_hbm, buf, sem, stage_sem):
    wid = lax.axis_index("s") * _NC + lax.axis_index("c")
    base = wid * _BPW
    stage = [pltpu.async_copy(table_hbm, buf.at[r], stage_sem) for r in range(_REP)]
    for c in stage:
        c.wait()
    copies = []
    for i in range(_NFULL):
        copies.append(
            pltpu.async_copy(buf, out_hbm.at[pl.ds(base + i * _REP, _REP)], sem)
        )
    if _TAIL:
        copies.append(
            pltpu.async_copy(
                buf.at[pl.ds(0, _TAIL)],
                out_hbm.at[pl.ds(base + _NFULL * _REP, _TAIL)],
                sem,
            )
        )
    for c in copies:
        c.wait()


def kernel(embedding_weight, batch_size):
    del batch_size
    return _broadcast_sc(embedding_weight)


# speedup vs baseline: 1.6188x; 1.5986x over previous
import functools

import jax
import jax.numpy as jnp
from jax.experimental import pallas as pl
from jax.experimental.pallas import tpu as pltpu

_NUM_BANDS = 64
_EMBED_DIM = 128
_B = 4096
_BLOCK_B = 256


def _body(table_ref, out_ref):
    out_ref[...] = jnp.broadcast_to(
        table_ref[...][None], (_BLOCK_B, _NUM_BANDS, _EMBED_DIM)
    )


@jax.jit
def _broadcast_tc(table):
    return pl.pallas_call(
        _body,
        grid=(_B // _BLOCK_B,),
        in_specs=[
            pl.BlockSpec((_NUM_BANDS, _EMBED_DIM), lambda i: (0, 0)),
        ],
        out_specs=pl.BlockSpec(
            (_BLOCK_B, _NUM_BANDS, _EMBED_DIM), lambda i: (i, 0, 0)
        ),
        out_shape=jax.ShapeDtypeStruct((_B, _NUM_BANDS, _EMBED_DIM), jnp.float32),
    )(table)


def kernel(embedding_weight, batch_size):
    del batch_size
    return _broadcast_tc(embedding_weight)
